# baseline (speedup 1.0000x reference)
import functools

import jax
import jax.numpy as jnp
from jax import lax
from jax.experimental import pallas as pl
from jax.experimental.pallas import tpu as pltpu
from jax.experimental.pallas import tpu_sc as plsc

V = 10000
N = 32
R = 5
A = 8
P = R * A
L = 16
VPT = 320
NG = VPT // L
NROW = 3 * P


def _body(tmpl_hbm, proj_hbm, w_hbm, i_hbm, tmpl_v, pbuf, ow, oi, sem):
    cid = lax.axis_index("c")
    sid = lax.axis_index("s")
    wid = sid * 2 + cid
    v0 = jnp.minimum(wid * VPT, V - VPT)

    pltpu.sync_copy(tmpl_hbm, tmpl_v)

    in_copies = [
        pltpu.async_copy(proj_hbm.at[pl.ds(row * V + v0, VPT)],
                         pbuf.at[pl.ds(row * VPT, VPT)], sem)
        for row in range(2 * N)
    ]
    for c in in_copies:
        c.wait()

    iota = lax.iota(jnp.int32, L)
    inf = jnp.full((L,), jnp.inf, jnp.float32)
    zero_i = jnp.zeros((L,), jnp.int32)

    def group_body(g, carry):
        goff = g * L
        vloc = goff + iota

        def p_body(p, carry_p):
            tv = tmpl_v[pl.ds(2 * p, L)]
            txv = jnp.full((L,), tv[0])
            tyv = jnp.full((L,), tv[1])

            d0 = d1 = d2 = inf
            i0 = i1 = i2 = zero_i
            for n in range(N):
                pxn = pbuf[pl.ds((2 * n) * VPT + goff, L)]
                pyn = pbuf[pl.ds((2 * n + 1) * VPT + goff, L)]
                dx = txv - pxn
                dy = tyv - pyn
                d = dx * dx + dy * dy
                nv = jnp.full((L,), n, jnp.int32)
                c0 = d < d0
                c1 = d < d1
                c2 = d < d2
                i2n = jnp.where(c1, i1, jnp.where(c2, nv, i2))
                d2n = jnp.where(c1, d1, jnp.where(c2, d, d2))
                i1n = jnp.where(c0, i0, jnp.where(c1, nv, i1))
                d1n = jnp.where(c0, d0, jnp.where(c1, d, d1))
                i0n = jnp.where(c0, nv, i0)
                d0n = jnp.where(c0, d, d0)
                d0, d1, d2, i0, i1, i2 = d0n, d1n, d2n, i0n, i1n, i2n

            gx0 = i0 * (2 * VPT) + vloc
            gx1 = i1 * (2 * VPT) + vloc
            gx2 = i2 * (2 * VPT) + vloc
            x0 = plsc.load_gather(pbuf, [gx0])
            y0 = plsc.load_gather(pbuf, [gx0 + VPT])
            x1 = plsc.load_gather(pbuf, [gx1])
            y1 = plsc.load_gather(pbuf, [gx1 + VPT])
            x2 = plsc.load_gather(pbuf, [gx2])
            y2 = plsc.load_gather(pbuf, [gx2 + VPT])

            v0x = x2 - x0
            v0y = y2 - y0
            v1x = x1 - x0
            v1y = y1 - y0
            v2x = txv - x0
            v2y = tyv - y0
            dot00 = v0x * v0x + v0y * v0y
            dot01 = v0x * v1x + v0y * v1y
            dot02 = v0x * v2x + v0y * v2y
            dot11 = v1x * v1x + v1y * v1y
            dot12 = v1x * v2x + v1y * v2y
            den = dot00 * dot11 - dot01 * dot01 + 1e-6
            w2 = (dot11 * dot02 - dot01 * dot12) / den
            w1 = (dot00 * dot12 - dot01 * dot02) / den
            w0 = 1.0 - w2 - w1

            r = p // A
            a = p % A
            wrow = r * (3 * A) + a
            ow[pl.ds(wrow * VPT + goff, L)] = w2
            ow[pl.ds((wrow + A) * VPT + goff, L)] = w1
            ow[pl.ds((wrow + 2 * A) * VPT + goff, L)] = w0
            oi[pl.ds(p * VPT + goff, L)] = i0
            oi[pl.ds((P + p) * VPT + goff, L)] = i1
            oi[pl.ds((2 * P + p) * VPT + goff, L)] = i2
            return carry_p

        lax.fori_loop(0, P, p_body, 0, unroll=2)
        return carry

    lax.fori_loop(0, NG, group_body, 0)

    out_copies = [
        pltpu.async_copy(ow.at[pl.ds(row * VPT, VPT)],
                         w_hbm.at[pl.ds(row * V + v0, VPT)], sem)
        for row in range(NROW)
    ] + [
        pltpu.async_copy(oi.at[pl.ds(row * VPT, VPT)],
                         i_hbm.at[pl.ds(row * V + v0, VPT)], sem)
        for row in range(NROW)
    ]
    for c in out_copies:
        c.wait()


@functools.cache
def _build():
    mesh = plsc.VectorSubcoreMesh(core_axis_name="c", subcore_axis_name="s")
    return functools.partial(
        pl.kernel,
        mesh=mesh,
        compiler_params=pltpu.CompilerParams(needs_layout_passes=False),
        out_type=(jax.ShapeDtypeStruct((NROW * V,), jnp.float32),
                  jax.ShapeDtypeStruct((NROW * V,), jnp.int32)),
        scratch_types=[
            pltpu.VMEM((8 * L,), jnp.float32),
            pltpu.VMEM((2 * N * VPT,), jnp.float32),
            pltpu.VMEM((NROW * VPT,), jnp.float32),
            pltpu.VMEM((NROW * VPT,), jnp.int32),
            pltpu.SemaphoreType.DMA,
        ],
    )(_body)


def kernel(template, projections):
    tmpl_flat = jnp.zeros((8 * L,), jnp.float32).at[:2 * P].set(
        template.reshape(-1))
    proj_t = jnp.transpose(projections, (1, 2, 0)).reshape(-1)
    w_lin, i_lin = _build()(tmpl_flat, proj_t)
    w = w_lin.reshape(R, 3, A, V).transpose(3, 0, 2, 1)
    ci = i_lin.reshape(3, R, A, V).transpose(3, 0, 1, 2)
    return w, ci


# speedup vs baseline: 1.3765x; 1.3765x over previous
import functools

import jax
import jax.numpy as jnp
from jax import lax
from jax.experimental import pallas as pl
from jax.experimental.pallas import tpu as pltpu
from jax.experimental.pallas import tpu_sc as plsc

V = 10000
N = 32
R = 5
A = 8
P = R * A
L = 16
NROW = 3 * P

VSC = 6144
VPT = VSC // 32
NG = VPT // L

VTCB = 1024
NTCB = (V - VSC + VTCB - 1) // VTCB
VTC = NTCB * VTCB
VS = VSC + VTC


def _top3_insert(txv, tyv, pxn, pyn, n, st):
    d0, d1, d2, i0, i1, i2 = st
    dx = txv - pxn
    dy = tyv - pyn
    d = dx * dx + dy * dy
    nv = jnp.full_like(i0, n)
    c0 = d < d0
    c1 = d < d1
    c2 = d < d2
    i2n = jnp.where(c1, i1, jnp.where(c2, nv, i2))
    d2n = jnp.where(c1, d1, jnp.where(c2, d, d2))
    i1n = jnp.where(c0, i0, jnp.where(c1, nv, i1))
    d1n = jnp.where(c0, d0, jnp.where(c1, d, d1))
    i0n = jnp.where(c0, nv, i0)
    d0n = jnp.where(c0, d, d0)
    return (d0n, d1n, d2n, i0n, i1n, i2n)


def _weights(txv, tyv, x0, y0, x1, y1, x2, y2):
    v0x = x2 - x0
    v0y = y2 - y0
    v1x = x1 - x0
    v1y = y1 - y0
    v2x = txv - x0
    v2y = tyv - y0
    dot00 = v0x * v0x + v0y * v0y
    dot01 = v0x * v1x + v0y * v1y
    dot02 = v0x * v2x + v0y * v2y
    dot11 = v1x * v1x + v1y * v1y
    dot12 = v1x * v2x + v1y * v2y
    den = dot00 * dot11 - dot01 * dot01 + 1e-6
    w2 = (dot11 * dot02 - dot01 * dot12) / den
    w1 = (dot00 * dot12 - dot01 * dot02) / den
    w0 = 1.0 - w2 - w1
    return w2, w1, w0


def _sc_body(tmpl_hbm, proj_hbm, w_hbm, i_hbm, tmpl_v, pbuf, ow, oi, sem):
    cid = lax.axis_index("c")
    sid = lax.axis_index("s")
    wid = sid * 2 + cid
    v0 = wid * VPT

    pltpu.sync_copy(tmpl_hbm, tmpl_v)

    in_copies = [
        pltpu.async_copy(proj_hbm.at[pl.ds(row * VS + v0, VPT)],
                         pbuf.at[pl.ds(row * VPT, VPT)], sem)
        for row in range(2 * N)
    ]
    for c in in_copies:
        c.wait()

    iota = lax.iota(jnp.int32, L)
    inf = jnp.full((L,), jnp.inf, jnp.float32)
    zero_i = jnp.zeros((L,), jnp.int32)

    def group_body(g, carry):
        goff = g * L
        vloc = goff + iota

        def p_body(p, carry_p):
            tv = tmpl_v[pl.ds(2 * p, L)]
            txv = jnp.full((L,), tv[0])
            tyv = jnp.full((L,), tv[1])

            st = (inf, inf, inf, zero_i, zero_i, zero_i)
            for n in range(N):
                pxn = pbuf[pl.ds((2 * n) * VPT + goff, L)]
                pyn = pbuf[pl.ds((2 * n + 1) * VPT + goff, L)]
                st = _top3_insert(txv, tyv, pxn, pyn, n, st)
            _, _, _, i0, i1, i2 = st

            gx0 = i0 * (2 * VPT) + vloc
            gx1 = i1 * (2 * VPT) + vloc
            gx2 = i2 * (2 * VPT) + vloc
            x0 = plsc.load_gather(pbuf, [gx0])
            y0 = plsc.load_gather(pbuf, [gx0 + VPT])
            x1 = plsc.load_gather(pbuf, [gx1])
            y1 = plsc.load_gather(pbuf, [gx1 + VPT])
            x2 = plsc.load_gather(pbuf, [gx2])
            y2 = plsc.load_gather(pbuf, [gx2 + VPT])

            w2, w1, w0 = _weights(txv, tyv, x0, y0, x1, y1, x2, y2)

            r = p // A
            a = p % A
            wrow = r * (3 * A) + a
            ow[pl.ds(wrow * VPT + goff, L)] = w2
            ow[pl.ds((wrow + A) * VPT + goff, L)] = w1
            ow[pl.ds((wrow + 2 * A) * VPT + goff, L)] = w0
            oi[pl.ds(p * VPT + goff, L)] = i0
            oi[pl.ds((P + p) * VPT + goff, L)] = i1
            oi[pl.ds((2 * P + p) * VPT + goff, L)] = i2
            return carry_p

        lax.fori_loop(0, P, p_body, 0)
        return carry

    lax.fori_loop(0, NG, group_body, 0)

    out_copies = [
        pltpu.async_copy(ow.at[pl.ds(row * VPT, VPT)],
                         w_hbm.at[pl.ds(row * VSC + v0, VPT)], sem)
        for row in range(NROW)
    ] + [
        pltpu.async_copy(oi.at[pl.ds(row * VPT, VPT)],
                         i_hbm.at[pl.ds(row * VSC + v0, VPT)], sem)
        for row in range(NROW)
    ]
    for c in out_copies:
        c.wait()


@functools.cache
def _build_sc():
    mesh = plsc.VectorSubcoreMesh(core_axis_name="c", subcore_axis_name="s")
    return functools.partial(
        pl.kernel,
        mesh=mesh,
        compiler_params=pltpu.CompilerParams(needs_layout_passes=False),
        out_type=(jax.ShapeDtypeStruct((NROW * VSC,), jnp.float32),
                  jax.ShapeDtypeStruct((NROW * VSC,), jnp.int32)),
        scratch_types=[
            pltpu.VMEM((8 * L,), jnp.float32),
            pltpu.VMEM((2 * N * VPT,), jnp.float32),
            pltpu.VMEM((NROW * VPT,), jnp.float32),
            pltpu.VMEM((NROW * VPT,), jnp.int32),
            pltpu.SemaphoreType.DMA,
        ],
    )(_sc_body)


def _tc_body(tmpl_ref, proj_ref, ow_ref, oi_ref):
    inf = jnp.full((8, 128), jnp.inf, jnp.float32)
    zero_i = jnp.zeros((8, 128), jnp.int32)

    def p_body(p, carry_p):
        txv = jnp.full((8, 128), tmpl_ref[2 * p])
        tyv = jnp.full((8, 128), tmpl_ref[2 * p + 1])

        st = (inf, inf, inf, zero_i, zero_i, zero_i)
        for n in range(N):
            pxn = proj_ref[2 * n]
            pyn = proj_ref[2 * n + 1]
            st = _top3_insert(txv, tyv, pxn, pyn, n, st)
        _, _, _, i0, i1, i2 = st

        x0 = y0 = x1 = y1 = x2 = y2 = jnp.zeros((8, 128), jnp.float32)
        for n in range(N):
            pxn = proj_ref[2 * n]
            pyn = proj_ref[2 * n + 1]
            nv = jnp.full((8, 128), n, jnp.int32)
            m0 = i0 == nv
            m1 = i1 == nv
            m2 = i2 == nv
            x0 = jnp.where(m0, pxn, x0)
            y0 = jnp.where(m0, pyn, y0)
            x1 = jnp.where(m1, pxn, x1)
            y1 = jnp.where(m1, pyn, y1)
            x2 = jnp.where(m2, pxn, x2)
            y2 = jnp.where(m2, pyn, y2)

        w2, w1, w0 = _weights(txv, tyv, x0, y0, x1, y1, x2, y2)

        r = p // A
        a = p % A
        wrow = r * (3 * A) + a
        ow_ref[pl.ds(wrow, 1)] = w2[None]
        ow_ref[pl.ds(wrow + A, 1)] = w1[None]
        ow_ref[pl.ds(wrow + 2 * A, 1)] = w0[None]
        oi_ref[pl.ds(p, 1)] = i0[None]
        oi_ref[pl.ds(P + p, 1)] = i1[None]
        oi_ref[pl.ds(2 * P + p, 1)] = i2[None]
        return carry_p

    lax.fori_loop(0, P, p_body, 0)


@functools.cache
def _build_tc():
    return pl.pallas_call(
        _tc_body,
        grid=(NTCB,),
        in_specs=[
            pl.BlockSpec(memory_space=pltpu.SMEM),
            pl.BlockSpec((2 * N, 8, 128), lambda j: (0, VSC // VTCB + j, 0)),
        ],
        out_specs=[
            pl.BlockSpec((NROW, 8, 128), lambda j: (0, j, 0)),
            pl.BlockSpec((NROW, 8, 128), lambda j: (0, j, 0)),
        ],
        out_shape=(jax.ShapeDtypeStruct((NROW, NTCB * 8, 128), jnp.float32),
                   jax.ShapeDtypeStruct((NROW, NTCB * 8, 128), jnp.int32)),
    )


def kernel(template, projections):
    tmpl_flat = jnp.zeros((8 * L,), jnp.float32).at[:2 * P].set(
        template.reshape(-1))
    proj_t = jnp.transpose(projections, (1, 2, 0)).reshape(2 * N, V)
    proj_tp = jnp.pad(proj_t, ((0, 0), (0, VS - V)))
    w_sc, i_sc = _build_sc()(tmpl_flat, proj_tp.reshape(-1))
    w_tc, i_tc = _build_tc()(tmpl_flat,
                             proj_tp.reshape(2 * N, VS // 128, 128))
    w_rows = jnp.concatenate(
        [w_sc.reshape(NROW, VSC),
         w_tc.reshape(NROW, VTC)[:, :V - VSC]], axis=1)
    i_rows = jnp.concatenate(
        [i_sc.reshape(NROW, VSC),
         i_tc.reshape(NROW, VTC)[:, :V - VSC]], axis=1)
    w = w_rows.reshape(R, 3, A, V).transpose(3, 0, 2, 1)
    ci = i_rows.reshape(3, R, A, V).transpose(3, 0, 1, 2)
    return w, ci


# speedup vs baseline: 1.5553x; 1.1298x over previous
import functools

import jax
import jax.numpy as jnp
from jax import lax
from jax.experimental import pallas as pl
from jax.experimental.pallas import tpu as pltpu
from jax.experimental.pallas import tpu_sc as plsc

V = 10000
N = 32
R = 5
A = 8
P = R * A
L = 16
NROW = 3 * P

VSC = 5120
VPT = VSC // 32
NG = VPT // L

VTCB = 1024
NTCB = (V - VSC + VTCB - 1) // VTCB
VTC = NTCB * VTCB
VS = VSC + VTC


def _top3_insert(txv, tyv, pxn, pyn, n, st):
    d0, d1, d2, i0, i1, i2 = st
    dx = txv - pxn
    dy = tyv - pyn
    d = dx * dx + dy * dy
    nv = jnp.full_like(i0, n)
    c0 = d < d0
    c1 = d < d1
    c2 = d < d2
    i2n = jnp.where(c1, i1, jnp.where(c2, nv, i2))
    d2n = jnp.where(c1, d1, jnp.where(c2, d, d2))
    i1n = jnp.where(c0, i0, jnp.where(c1, nv, i1))
    d1n = jnp.where(c0, d0, jnp.where(c1, d, d1))
    i0n = jnp.where(c0, nv, i0)
    d0n = jnp.where(c0, d, d0)
    return (d0n, d1n, d2n, i0n, i1n, i2n)


def _weights(txv, tyv, x0, y0, x1, y1, x2, y2):
    v0x = x2 - x0
    v0y = y2 - y0
    v1x = x1 - x0
    v1y = y1 - y0
    v2x = txv - x0
    v2y = tyv - y0
    dot00 = v0x * v0x + v0y * v0y
    dot01 = v0x * v1x + v0y * v1y
    dot02 = v0x * v2x + v0y * v2y
    dot11 = v1x * v1x + v1y * v1y
    dot12 = v1x * v2x + v1y * v2y
    den = dot00 * dot11 - dot01 * dot01 + 1e-6
    w2 = (dot11 * dot02 - dot01 * dot12) / den
    w1 = (dot00 * dot12 - dot01 * dot02) / den
    w0 = 1.0 - w2 - w1
    return w2, w1, w0


def _sc_body(tmpl_hbm, proj_hbm, w_hbm, i_hbm, tmpl_v, pbuf, ow, oi, sem):
    cid = lax.axis_index("c")
    sid = lax.axis_index("s")
    wid = sid * 2 + cid
    v0 = wid * VPT

    pltpu.sync_copy(tmpl_hbm, tmpl_v)

    in_copies = [
        pltpu.async_copy(proj_hbm.at[pl.ds(row * VS + v0, VPT)],
                         pbuf.at[pl.ds(row * VPT, VPT)], sem)
        for row in range(2 * N)
    ]
    for c in in_copies:
        c.wait()

    iota = lax.iota(jnp.int32, L)
    inf = jnp.full((L,), jnp.inf, jnp.float32)
    zero_i = jnp.zeros((L,), jnp.int32)

    def group_body(g, carry):
        goff = g * L
        vloc = goff + iota

        def p_body(p, carry_p):
            tv = tmpl_v[pl.ds(2 * p, L)]
            txv = jnp.full((L,), tv[0])
            tyv = jnp.full((L,), tv[1])

            st = (inf, inf, inf, zero_i, zero_i, zero_i)
            for n in range(N):
                pxn = pbuf[pl.ds((2 * n) * VPT + goff, L)]
                pyn = pbuf[pl.ds((2 * n + 1) * VPT + goff, L)]
                st = _top3_insert(txv, tyv, pxn, pyn, n, st)
            _, _, _, i0, i1, i2 = st

            gx0 = i0 * (2 * VPT) + vloc
            gx1 = i1 * (2 * VPT) + vloc
            gx2 = i2 * (2 * VPT) + vloc
            x0 = plsc.load_gather(pbuf, [gx0])
            y0 = plsc.load_gather(pbuf, [gx0 + VPT])
            x1 = plsc.load_gather(pbuf, [gx1])
            y1 = plsc.load_gather(pbuf, [gx1 + VPT])
            x2 = plsc.load_gather(pbuf, [gx2])
            y2 = plsc.load_gather(pbuf, [gx2 + VPT])

            w2, w1, w0 = _weights(txv, tyv, x0, y0, x1, y1, x2, y2)

            r = p // A
            a = p % A
            wrow = r * (3 * A) + a
            ow[pl.ds(wrow * VPT + goff, L)] = w2
            ow[pl.ds((wrow + A) * VPT + goff, L)] = w1
            ow[pl.ds((wrow + 2 * A) * VPT + goff, L)] = w0
            oi[pl.ds(p * VPT + goff, L)] = i0
            oi[pl.ds((P + p) * VPT + goff, L)] = i1
            oi[pl.ds((2 * P + p) * VPT + goff, L)] = i2
            return carry_p

        lax.fori_loop(0, P, p_body, 0)
        return carry

    lax.fori_loop(0, NG, group_body, 0)

    out_copies = [
        pltpu.async_copy(ow.at[pl.ds(row * VPT, VPT)],
                         w_hbm.at[pl.ds(row * VSC + v0, VPT)], sem)
        for row in range(NROW)
    ] + [
        pltpu.async_copy(oi.at[pl.ds(row * VPT, VPT)],
                         i_hbm.at[pl.ds(row * VSC + v0, VPT)], sem)
        for row in range(NROW)
    ]
    for c in out_copies:
        c.wait()


@functools.cache
def _build_sc():
    mesh = plsc.VectorSubcoreMesh(core_axis_name="c", subcore_axis_name="s")
    return functools.partial(
        pl.kernel,
        mesh=mesh,
        compiler_params=pltpu.CompilerParams(needs_layout_passes=False),
        out_type=(jax.ShapeDtypeStruct((NROW * VSC,), jnp.float32),
                  jax.ShapeDtypeStruct((NROW * VSC,), jnp.int32)),
        scratch_types=[
            pltpu.VMEM((8 * L,), jnp.float32),
            pltpu.VMEM((2 * N * VPT,), jnp.float32),
            pltpu.VMEM((NROW * VPT,), jnp.float32),
            pltpu.VMEM((NROW * VPT,), jnp.int32),
            pltpu.SemaphoreType.DMA,
        ],
    )(_sc_body)


def _tc_body(tmpl_ref, proj_ref, ow_ref, oi_ref):
    inf = jnp.full((8, 128), jnp.inf, jnp.float32)
    zero_i = jnp.zeros((8, 128), jnp.int32)

    def p_body(p, carry_p):
        txv = jnp.full((8, 128), tmpl_ref[2 * p])
        tyv = jnp.full((8, 128), tmpl_ref[2 * p + 1])

        st = (inf, inf, inf, zero_i, zero_i, zero_i)
        for n in range(N):
            pxn = proj_ref[2 * n]
            pyn = proj_ref[2 * n + 1]
            st = _top3_insert(txv, tyv, pxn, pyn, n, st)
        _, _, _, i0, i1, i2 = st

        x0 = y0 = x1 = y1 = x2 = y2 = jnp.zeros((8, 128), jnp.float32)
        for n in range(N):
            pxn = proj_ref[2 * n]
            pyn = proj_ref[2 * n + 1]
            nv = jnp.full((8, 128), n, jnp.int32)
            m0 = i0 == nv
            m1 = i1 == nv
            m2 = i2 == nv
            x0 = jnp.where(m0, pxn, x0)
            y0 = jnp.where(m0, pyn, y0)
            x1 = jnp.where(m1, pxn, x1)
            y1 = jnp.where(m1, pyn, y1)
            x2 = jnp.where(m2, pxn, x2)
            y2 = jnp.where(m2, pyn, y2)

        w2, w1, w0 = _weights(txv, tyv, x0, y0, x1, y1, x2, y2)

        r = p // A
        a = p % A
        wrow = r * (3 * A) + a
        ow_ref[pl.ds(wrow, 1)] = w2[None]
        ow_ref[pl.ds(wrow + A, 1)] = w1[None]
        ow_ref[pl.ds(wrow + 2 * A, 1)] = w0[None]
        oi_ref[pl.ds(p, 1)] = i0[None]
        oi_ref[pl.ds(P + p, 1)] = i1[None]
        oi_ref[pl.ds(2 * P + p, 1)] = i2[None]
        return carry_p

    lax.fori_loop(0, P, p_body, 0)


@functools.cache
def _build_tc():
    return pl.pallas_call(
        _tc_body,
        grid=(NTCB,),
        in_specs=[
            pl.BlockSpec(memory_space=pltpu.SMEM),
            pl.BlockSpec((2 * N, 8, 128), lambda j: (0, VSC // VTCB + j, 0)),
        ],
        out_specs=[
            pl.BlockSpec((NROW, 8, 128), lambda j: (0, j, 0)),
            pl.BlockSpec((NROW, 8, 128), lambda j: (0, j, 0)),
        ],
        out_shape=(jax.ShapeDtypeStruct((NROW, NTCB * 8, 128), jnp.float32),
                   jax.ShapeDtypeStruct((NROW, NTCB * 8, 128), jnp.int32)),
    )


def kernel(template, projections):
    tmpl_flat = jnp.zeros((8 * L,), jnp.float32).at[:2 * P].set(
        template.reshape(-1))
    proj_t = jnp.transpose(projections, (1, 2, 0)).reshape(2 * N, V)
    proj_tp = jnp.pad(proj_t, ((0, 0), (0, VS - V)))
    w_sc, i_sc = _build_sc()(tmpl_flat, proj_tp.reshape(-1))
    w_tc, i_tc = _build_tc()(tmpl_flat,
                             proj_tp.reshape(2 * N, VS // 128, 128))
    w_rows = jnp.concatenate(
        [w_sc.reshape(NROW, VSC),
         w_tc.reshape(NROW, VTC)[:, :V - VSC]], axis=1)
    i_rows = jnp.concatenate(
        [i_sc.reshape(NROW, VSC),
         i_tc.reshape(NROW, VTC)[:, :V - VSC]], axis=1)
    w = w_rows.reshape(R, 3, A, V).transpose(3, 0, 2, 1)
    ci = i_rows.reshape(3, R, A, V).transpose(3, 0, 1, 2)
    return w, ci


# speedup vs baseline: 1.5953x; 1.0258x over previous
import functools

import jax
import jax.numpy as jnp
from jax import lax
from jax.experimental import pallas as pl
from jax.experimental.pallas import tpu as pltpu
from jax.experimental.pallas import tpu_sc as plsc

V = 10000
N = 32
R = 5
A = 8
P = R * A
L = 16
NROW = 3 * P

VSC = 5120
VPT = VSC // 32
NG = VPT // L

VTCB = 1024
NTCB = (V - VSC + VTCB - 1) // VTCB
VTC = NTCB * VTCB
VS = VSC + VTC


def _top3_insert(txv, tyv, pxn, pyn, n, st):
    d0, d1, d2, i0, i1, i2 = st
    dx = txv - pxn
    dy = tyv - pyn
    d = dx * dx + dy * dy
    nv = jnp.full_like(i0, n)
    c0 = d < d0
    c1 = d < d1
    c2 = d < d2
    i2n = jnp.where(c1, i1, jnp.where(c2, nv, i2))
    d2n = jnp.where(c1, d1, jnp.where(c2, d, d2))
    i1n = jnp.where(c0, i0, jnp.where(c1, nv, i1))
    d1n = jnp.where(c0, d0, jnp.where(c1, d, d1))
    i0n = jnp.where(c0, nv, i0)
    d0n = jnp.where(c0, d, d0)
    return (d0n, d1n, d2n, i0n, i1n, i2n)


def _weights(txv, tyv, x0, y0, x1, y1, x2, y2):
    v0x = x2 - x0
    v0y = y2 - y0
    v1x = x1 - x0
    v1y = y1 - y0
    v2x = txv - x0
    v2y = tyv - y0
    dot00 = v0x * v0x + v0y * v0y
    dot01 = v0x * v1x + v0y * v1y
    dot02 = v0x * v2x + v0y * v2y
    dot11 = v1x * v1x + v1y * v1y
    dot12 = v1x * v2x + v1y * v2y
    den = dot00 * dot11 - dot01 * dot01 + 1e-6
    w2 = (dot11 * dot02 - dot01 * dot12) / den
    w1 = (dot00 * dot12 - dot01 * dot02) / den
    w0 = 1.0 - w2 - w1
    return w2, w1, w0


def _sc_body(tmpl_hbm, proj_hbm, w_hbm, i_hbm, tmpl_v, pbuf, ow, oi, sem):
    cid = lax.axis_index("c")
    sid = lax.axis_index("s")
    wid = sid * 2 + cid
    v0 = wid * VPT

    pltpu.sync_copy(tmpl_hbm, tmpl_v)

    def in_fire(row, carry):
        pltpu.async_copy(proj_hbm.at[pl.ds(row * VS + v0, VPT)],
                         pbuf.at[pl.ds(row * VPT, VPT)], sem)
        return carry

    lax.fori_loop(0, 2 * N, in_fire, 0)
    pltpu.make_async_copy(proj_hbm.at[pl.ds(0, 2 * N * VPT)], pbuf,
                          sem).wait()

    iota = lax.iota(jnp.int32, L)
    inf = jnp.full((L,), jnp.inf, jnp.float32)
    zero_i = jnp.zeros((L,), jnp.int32)

    def group_body(g, carry):
        goff = g * L
        vloc = goff + iota

        def p_body(p, carry_p):
            tv = tmpl_v[pl.ds(2 * p, L)]
            txv = jnp.full((L,), tv[0])
            tyv = jnp.full((L,), tv[1])

            st = (inf, inf, inf, zero_i, zero_i, zero_i)
            for n in range(N):
                pxn = pbuf[pl.ds((2 * n) * VPT + goff, L)]
                pyn = pbuf[pl.ds((2 * n + 1) * VPT + goff, L)]
                st = _top3_insert(txv, tyv, pxn, pyn, n, st)
            _, _, _, i0, i1, i2 = st

            gx0 = i0 * (2 * VPT) + vloc
            gx1 = i1 * (2 * VPT) + vloc
            gx2 = i2 * (2 * VPT) + vloc
            x0 = plsc.load_gather(pbuf, [gx0])
            y0 = plsc.load_gather(pbuf, [gx0 + VPT])
            x1 = plsc.load_gather(pbuf, [gx1])
            y1 = plsc.load_gather(pbuf, [gx1 + VPT])
            x2 = plsc.load_gather(pbuf, [gx2])
            y2 = plsc.load_gather(pbuf, [gx2 + VPT])

            w2, w1, w0 = _weights(txv, tyv, x0, y0, x1, y1, x2, y2)

            r = p // A
            a = p % A
            wrow = r * (3 * A) + a
            ow[pl.ds(wrow * VPT + goff, L)] = w2
            ow[pl.ds((wrow + A) * VPT + goff, L)] = w1
            ow[pl.ds((wrow + 2 * A) * VPT + goff, L)] = w0
            oi[pl.ds(p * VPT + goff, L)] = i0
            oi[pl.ds((P + p) * VPT + goff, L)] = i1
            oi[pl.ds((2 * P + p) * VPT + goff, L)] = i2
            return carry_p

        lax.fori_loop(0, P, p_body, 0)
        return carry

    lax.fori_loop(0, NG, group_body, 0)

    def out_fire(row, carry):
        pltpu.async_copy(ow.at[pl.ds(row * VPT, VPT)],
                         w_hbm.at[pl.ds(row * VSC + v0, VPT)], sem)
        pltpu.async_copy(oi.at[pl.ds(row * VPT, VPT)],
                         i_hbm.at[pl.ds(row * VSC + v0, VPT)], sem)
        return carry

    lax.fori_loop(0, NROW, out_fire, 0)
    pltpu.make_async_copy(w_hbm.at[pl.ds(0, NROW * VPT)], ow, sem).wait()
    pltpu.make_async_copy(i_hbm.at[pl.ds(0, NROW * VPT)], oi, sem).wait()


@functools.cache
def _build_sc():
    mesh = plsc.VectorSubcoreMesh(core_axis_name="c", subcore_axis_name="s")
    return functools.partial(
        pl.kernel,
        mesh=mesh,
        compiler_params=pltpu.CompilerParams(needs_layout_passes=False),
        out_type=(jax.ShapeDtypeStruct((NROW * VSC,), jnp.float32),
                  jax.ShapeDtypeStruct((NROW * VSC,), jnp.int32)),
        scratch_types=[
            pltpu.VMEM((8 * L,), jnp.float32),
            pltpu.VMEM((2 * N * VPT,), jnp.float32),
            pltpu.VMEM((NROW * VPT,), jnp.float32),
            pltpu.VMEM((NROW * VPT,), jnp.int32),
            pltpu.SemaphoreType.DMA,
        ],
    )(_sc_body)


def _tc_body(tmpl_ref, proj_ref, ow_ref, oi_ref):
    inf = jnp.full((8, 128), jnp.inf, jnp.float32)
    zero_i = jnp.zeros((8, 128), jnp.int32)

    def p_body(p, carry_p):
        txv = jnp.full((8, 128), tmpl_ref[2 * p])
        tyv = jnp.full((8, 128), tmpl_ref[2 * p + 1])

        st = (inf, inf, inf, zero_i, zero_i, zero_i)
        for n in range(N):
            pxn = proj_ref[2 * n]
            pyn = proj_ref[2 * n + 1]
            st = _top3_insert(txv, tyv, pxn, pyn, n, st)
        _, _, _, i0, i1, i2 = st

        x0 = y0 = x1 = y1 = x2 = y2 = jnp.zeros((8, 128), jnp.float32)
        for n in range(N):
            pxn = proj_ref[2 * n]
            pyn = proj_ref[2 * n + 1]
            nv = jnp.full((8, 128), n, jnp.int32)
            m0 = i0 == nv
            m1 = i1 == nv
            m2 = i2 == nv
            x0 = jnp.where(m0, pxn, x0)
            y0 = jnp.where(m0, pyn, y0)
            x1 = jnp.where(m1, pxn, x1)
            y1 = jnp.where(m1, pyn, y1)
            x2 = jnp.where(m2, pxn, x2)
            y2 = jnp.where(m2, pyn, y2)

        w2, w1, w0 = _weights(txv, tyv, x0, y0, x1, y1, x2, y2)

        r = p // A
        a = p % A
        wrow = r * (3 * A) + a
        ow_ref[pl.ds(wrow, 1)] = w2[None]
        ow_ref[pl.ds(wrow + A, 1)] = w1[None]
        ow_ref[pl.ds(wrow + 2 * A, 1)] = w0[None]
        oi_ref[pl.ds(p, 1)] = i0[None]
        oi_ref[pl.ds(P + p, 1)] = i1[None]
        oi_ref[pl.ds(2 * P + p, 1)] = i2[None]
        return carry_p

    lax.fori_loop(0, P, p_body, 0)


@functools.cache
def _build_tc():
    return pl.pallas_call(
        _tc_body,
        grid=(NTCB,),
        in_specs=[
            pl.BlockSpec(memory_space=pltpu.SMEM),
            pl.BlockSpec((2 * N, 8, 128), lambda j: (0, VSC // VTCB + j, 0)),
        ],
        out_specs=[
            pl.BlockSpec((NROW, 8, 128), lambda j: (0, j, 0)),
            pl.BlockSpec((NROW, 8, 128), lambda j: (0, j, 0)),
        ],
        out_shape=(jax.ShapeDtypeStruct((NROW, NTCB * 8, 128), jnp.float32),
                   jax.ShapeDtypeStruct((NROW, NTCB * 8, 128), jnp.int32)),
    )


def kernel(template, projections):
    tmpl_flat = jnp.zeros((8 * L,), jnp.float32).at[:2 * P].set(
        template.reshape(-1))
    proj_t = jnp.transpose(projections, (1, 2, 0)).reshape(2 * N, V)
    proj_tp = jnp.pad(proj_t, ((0, 0), (0, VS - V)))
    w_sc, i_sc = _build_sc()(tmpl_flat, proj_tp.reshape(-1))
    w_tc, i_tc = _build_tc()(tmpl_flat,
                             proj_tp.reshape(2 * N, VS // 128, 128))
    w_rows = jnp.concatenate(
        [w_sc.reshape(NROW, VSC),
         w_tc.reshape(NROW, VTC)[:, :V - VSC]], axis=1)
    i_rows = jnp.concatenate(
        [i_sc.reshape(NROW, VSC),
         i_tc.reshape(NROW, VTC)[:, :V - VSC]], axis=1)
    w = w_rows.reshape(R, 3, A, V).transpose(3, 0, 2, 1)
    ci = i_rows.reshape(3, R, A, V).transpose(3, 0, 1, 2)
    return w, ci
